# Initial kernel scaffold; baseline (speedup 1.0000x reference)
import functools

import jax
import jax.numpy as jnp
from jax import lax
from jax.experimental import pallas as pl
from jax.experimental.pallas import tpu as pltpu
from jax.experimental.pallas import tpu_sc as plsc

_D = 144
_GATHER_W = 128
_EDGE_BLK = 1600
_SCAT_C = 80
_NODE_BLK = 1024


def _gather_sc(table, idx_flat):
  m = idx_flat.shape[0]
  d = table.shape[1]
  mesh = plsc.VectorSubcoreMesh(core_axis_name="core",
                                subcore_axis_name="subcore")
  idx2 = idx_flat.reshape(1, m)

  @functools.partial(
      pl.kernel,
      out_type=jax.ShapeDtypeStruct((m, d), table.dtype),
      mesh=mesh)
  def k(t_hbm, i_hbm, o_hbm):
    def body(i_vmem, o_vmem):
      pltpu.sync_copy(t_hbm.at[i_vmem.at[0]], o_vmem)

    pltpu.emit_pipeline(
        body,
        grid=(m // _GATHER_W,),
        in_specs=[pl.BlockSpec((1, _GATHER_W), lambda i: (0, i))],
        out_specs=[pl.BlockSpec((_GATHER_W, d), lambda i: (i, 0))],
        core_axis_name=("core", "subcore"),
        dimension_semantics=(pltpu.PARALLEL,),
    )(i_hbm, o_hbm)

  return k(table, idx2)


def _scatter_add_sc(vec, col3, zeros_acc):
  n_pad = zeros_acc.shape[0]
  j_steps = col3.shape[1]
  c = col3.shape[2]
  rows_per_tile = n_pad // 16
  mesh = plsc.VectorSubcoreMesh(core_axis_name="core",
                                subcore_axis_name="subcore")

  @functools.partial(
      pl.kernel,
      out_type=jax.ShapeDtypeStruct((2, n_pad, 16), jnp.float32),
      mesh=mesh,
      scratch_types=[
          pltpu.VMEM((c,), jnp.int32),
          pltpu.VMEM((c, 16), jnp.float32),
          pltpu.VMEM_SHARED((n_pad, 16), jnp.float32),
      ])
  def k(vec_hbm, col_hbm, z_hbm, out_hbm, idx_v, vec_v, acc_sh):
    cid = lax.axis_index("core")
    sid = lax.axis_index("subcore")
    wid = cid * 16 + sid

    @pl.when(sid == 0)
    def _():
      pltpu.sync_copy(z_hbm, acc_sh)

    plsc.subcore_barrier()

    @pl.loop(0, j_steps)
    def _(j):
      base = (wid * j_steps + j) * c
      pltpu.sync_copy(col_hbm.at[wid, j], idx_v)
      pltpu.sync_copy(vec_hbm.at[pl.ds(base, c)], vec_v)
      pltpu.sync_copy(vec_v, acc_sh.at[idx_v], add=True)

    plsc.subcore_barrier()
    pltpu.sync_copy(acc_sh.at[pl.ds(sid * rows_per_tile, rows_per_tile)],
                    out_hbm.at[cid, pl.ds(sid * rows_per_tile, rows_per_tile)])

  return k(vec, col3, zeros_acc)


def _omega(dist):
  p = 5.0
  dr = jnp.clip(dist / 4.5, 0.0, 1.0)
  dr2 = dr * dr
  dr5 = dr2 * dr2 * dr
  dr6 = dr5 * dr
  dr7 = dr6 * dr
  out = (1.0
         - (p + 1.0) * (p + 2.0) / 2.0 * dr5
         + p * (p + 2.0) * dr6
         - p * (p + 1.0) / 2.0 * dr7)
  return jnp.clip(out, 0.0, 1.0)


def _mlp_body(gr, gc, wr, wc, wd, ba, wb, bb, out):
  xr = gr[:, :128]
  xc = gc[:, :128]
  pr = gr[:, 128:131]
  pc = gc[:, 128:131]
  d = pr - pc
  dist = jnp.sqrt(jnp.sum(d * d, axis=1, keepdims=True) + 1e-12)
  h = (jnp.dot(xr, wr[...], preferred_element_type=jnp.float32,
               precision=lax.Precision.HIGHEST)
       + jnp.dot(xc, wc[...], preferred_element_type=jnp.float32,
                 precision=lax.Precision.HIGHEST)
       + dist * wd[...] + ba[...])
  h = h * jax.nn.sigmoid(h)
  mes = jnp.dot(h, wb[...], preferred_element_type=jnp.float32,
                precision=lax.Precision.HIGHEST) + bb[...]
  scale = _omega(dist) / (dist + 1e-8)
  v1 = d * (scale * mes[:, 0:1])
  v2 = d * (scale * mes[:, 1:2])
  z = jnp.zeros((d.shape[0], 5), jnp.float32)
  out[...] = jnp.concatenate([v1, z, v2, z], axis=1)


def _mlp_tc(g, wr, wc, wd, ba, wb, bb, n_edges, interpret=False):
  nblk = n_edges // _EDGE_BLK
  return pl.pallas_call(
      _mlp_body,
      grid=(nblk,),
      in_specs=[
          pl.BlockSpec((_EDGE_BLK, _D), lambda i: (i, 0)),
          pl.BlockSpec((_EDGE_BLK, _D), lambda i, _n=nblk: (i + _n, 0)),
          pl.BlockSpec((128, 256), lambda i: (0, 0)),
          pl.BlockSpec((128, 256), lambda i: (0, 0)),
          pl.BlockSpec((1, 256), lambda i: (0, 0)),
          pl.BlockSpec((1, 256), lambda i: (0, 0)),
          pl.BlockSpec((256, 2), lambda i: (0, 0)),
          pl.BlockSpec((1, 2), lambda i: (0, 0)),
      ],
      out_specs=pl.BlockSpec((_EDGE_BLK, 16), lambda i: (i, 0)),
      out_shape=jax.ShapeDtypeStruct((n_edges, 16), jnp.float32),
      interpret=interpret,
  )(g, g, wr, wc, wd, ba, wb, bb)


def _gram_body(p0, p1, rnd, out):
  eps = 1e-6
  b = p0.shape[0]
  v = p0[...] + p1[...]
  v1 = v[:, 0:3]
  v2 = v[:, 8:11]

  def safe_norm(u):
    return jnp.sqrt(jnp.sum(u * u, axis=1, keepdims=True) + 1e-12)

  v1_norm = safe_norm(v1)
  mask1 = v1_norm > eps
  n1 = v1 / (v1_norm + eps)
  r = rnd[:, 0:3]
  r = r / safe_norm(r)
  n1 = jnp.where(mask1, n1, r)
  n2p = v2 - jnp.sum(n1 * v2, axis=1, keepdims=True) * n1
  n2_norm = safe_norm(n2p)
  mask2 = n2_norm > eps
  zc = jnp.zeros((b, 1), jnp.float32)
  oc = jnp.ones((b, 1), jnp.float32)
  default_orth = jnp.concatenate([-n1[:, 1:2], n1[:, 0:1], zc], axis=1)
  d_norm = safe_norm(default_orth)
  d_mask = d_norm > eps
  e3 = jnp.concatenate([zc, zc, oc], axis=1)
  default_orth = jnp.where(d_mask, default_orth / (d_norm + eps), e3)
  n2 = jnp.where(mask2, n2p / (n2_norm + eps), default_orth)
  n3 = jnp.concatenate([
      n1[:, 1:2] * n2[:, 2:3] - n1[:, 2:3] * n2[:, 1:2],
      n1[:, 2:3] * n2[:, 0:1] - n1[:, 0:1] * n2[:, 2:3],
      n1[:, 0:1] * n2[:, 1:2] - n1[:, 1:2] * n2[:, 0:1],
  ], axis=1)
  n3 = n3 / (safe_norm(n3) + eps)
  z7 = jnp.zeros((b, 7), jnp.float32)
  out[...] = jnp.concatenate([n1, n2, n3, z7], axis=1)


def _gram_tc(p0, p1, rnd16, interpret=False):
  n_pad = p0.shape[0]
  return pl.pallas_call(
      _gram_body,
      grid=(n_pad // _NODE_BLK,),
      in_specs=[
          pl.BlockSpec((_NODE_BLK, 16), lambda i: (i, 0)),
          pl.BlockSpec((_NODE_BLK, 16), lambda i: (i, 0)),
          pl.BlockSpec((_NODE_BLK, 16), lambda i: (i, 0)),
      ],
      out_specs=pl.BlockSpec((_NODE_BLK, 16), lambda i: (i, 0)),
      out_shape=jax.ShapeDtypeStruct((n_pad, 16), jnp.float32),
      interpret=interpret,
  )(p0, p1, rnd16)


def kernel(x, pos, edge_index, W1a, b1a, W1b, b1b, W2a, b2a, W2b, b2b):
  n, f = x.shape
  e = edge_index.shape[1]
  n_pad = ((n + 1023) // 1024) * 1024

  table = jnp.concatenate(
      [x, pos, jnp.zeros((n, _D - f - 3), jnp.float32)], axis=1)

  idx_flat = edge_index.reshape(2 * e)
  g = _gather_sc(table, idx_flat)

  wr = jnp.concatenate([W1a[:f], W2a[:f]], axis=1)
  wc = jnp.concatenate([W1a[f:2 * f], W2a[f:2 * f]], axis=1)
  wd = jnp.concatenate([W1a[2 * f:], W2a[2 * f:]], axis=1)
  ba = jnp.concatenate([b1a, b2a]).reshape(1, 256)
  wb = jnp.concatenate([
      jnp.concatenate([W1b, jnp.zeros_like(W1b)], axis=1),
      jnp.concatenate([jnp.zeros_like(W2b), W2b], axis=1),
  ], axis=0)
  bb = jnp.concatenate([b1b, b2b]).reshape(1, 2)

  vec = _mlp_tc(g, wr, wc, wd, ba, wb, bb, e)

  c = _SCAT_C
  j_steps = e // (32 * c)
  col3 = edge_index[1].reshape(32, j_steps, c)
  zeros_acc = jnp.zeros((n_pad, 16), jnp.float32)
  partials = _scatter_add_sc(vec, col3, zeros_acc)

  rnd = jax.random.normal(jax.random.key(42), (n, 3), dtype=jnp.float32)
  rnd16 = jnp.concatenate(
      [rnd, jnp.ones((n, 13), jnp.float32)], axis=1)
  rnd16 = jnp.concatenate(
      [rnd16, jnp.ones((n_pad - n, 16), jnp.float32)], axis=0)
  res = _gram_tc(partials[0], partials[1], rnd16)
  frames = res[:n, :9].reshape(n, 3, 3).transpose(0, 2, 1)
  return frames


# speedup vs baseline: 4.0784x; 4.0784x over previous
import functools

import jax
import jax.numpy as jnp
from jax import lax
from jax.experimental import pallas as pl
from jax.experimental.pallas import tpu as pltpu
from jax.experimental.pallas import tpu_sc as plsc

_D = 256
_GATHER_W = 128
_EDGE_BLK = 1600
_SCAT_C = 80
_NODE_BLK = 1024


def _gather_sc(table, idx_flat):
  m = idx_flat.shape[0]
  d = table.shape[1]
  mesh = plsc.VectorSubcoreMesh(core_axis_name="core",
                                subcore_axis_name="subcore")
  idx2 = idx_flat.reshape(1, m)

  @functools.partial(
      pl.kernel,
      out_type=jax.ShapeDtypeStruct((m, d), table.dtype),
      mesh=mesh)
  def k(t_hbm, i_hbm, o_hbm):
    def body(i_vmem, o_vmem):
      pltpu.sync_copy(t_hbm.at[i_vmem.at[0]], o_vmem)

    pltpu.emit_pipeline(
        body,
        grid=(m // _GATHER_W,),
        in_specs=[pl.BlockSpec((1, _GATHER_W), lambda i: (0, i))],
        out_specs=[pl.BlockSpec((_GATHER_W, d), lambda i: (i, 0))],
        core_axis_name=("core", "subcore"),
        dimension_semantics=(pltpu.PARALLEL,),
    )(i_hbm, o_hbm)

  return k(table, idx2)


def _scatter_add_sc(vec, col3, zeros_acc):
  n_pad = zeros_acc.shape[0]
  j_steps = col3.shape[1]
  c = col3.shape[2]
  rows_per_tile = n_pad // 16
  mesh = plsc.VectorSubcoreMesh(core_axis_name="core",
                                subcore_axis_name="subcore")

  @functools.partial(
      pl.kernel,
      out_type=jax.ShapeDtypeStruct((2, n_pad, 128), jnp.float32),
      mesh=mesh,
      scratch_types=[
          pltpu.VMEM((c,), jnp.int32),
          pltpu.VMEM((c, 128), jnp.float32),
          pltpu.VMEM_SHARED((n_pad, 128), jnp.float32),
      ])
  def k(vec_hbm, col_hbm, z_hbm, out_hbm, idx_v, vec_v, acc_sh):
    cid = lax.axis_index("core")
    sid = lax.axis_index("subcore")
    wid = cid * 16 + sid

    @pl.when(sid == 0)
    def _():
      pltpu.sync_copy(z_hbm, acc_sh)

    plsc.subcore_barrier()

    @pl.loop(0, j_steps)
    def _(j):
      base = (wid * j_steps + j) * c
      pltpu.sync_copy(col_hbm.at[wid, j], idx_v)
      pltpu.sync_copy(vec_hbm.at[pl.ds(base, c)], vec_v)
      pltpu.sync_copy(vec_v, acc_sh.at[idx_v], add=True)

    plsc.subcore_barrier()
    pltpu.sync_copy(acc_sh.at[pl.ds(sid * rows_per_tile, rows_per_tile)],
                    out_hbm.at[cid, pl.ds(sid * rows_per_tile, rows_per_tile)])

  return k(vec, col3, zeros_acc)


def _omega(dist):
  p = 5.0
  dr = jnp.clip(dist / 4.5, 0.0, 1.0)
  dr2 = dr * dr
  dr5 = dr2 * dr2 * dr
  dr6 = dr5 * dr
  dr7 = dr6 * dr
  out = (1.0
         - (p + 1.0) * (p + 2.0) / 2.0 * dr5
         + p * (p + 2.0) * dr6
         - p * (p + 1.0) / 2.0 * dr7)
  return jnp.clip(out, 0.0, 1.0)


def _mlp_body(gr, gc, wcat, ba, wb, bb, out):
  xr = gr[:, :128]
  xc = gc[:, :128]
  pr = gr[:, 128:131]
  pc = gc[:, 128:131]
  d = pr - pc
  dist = jnp.sqrt(jnp.sum(d * d, axis=1, keepdims=True) + 1e-12)
  b = d.shape[0]
  xcat = jnp.concatenate([xr, xc, dist, jnp.zeros((b, 127), jnp.float32)],
                         axis=1)
  h = jnp.dot(xcat, wcat[...], preferred_element_type=jnp.float32) + ba[...]
  h = jax.nn.silu(h)
  mes = jnp.dot(h, wb[...], preferred_element_type=jnp.float32) + bb[...]
  coe = _omega(dist)
  nv = d / (dist + 1e-8)
  v1 = nv * coe * mes[:, 0:1]
  v2 = nv * coe * mes[:, 1:2]
  z5 = jnp.zeros((b, 5), jnp.float32)
  z117 = jnp.zeros((b, 117), jnp.float32)
  out[...] = jnp.concatenate([v1, z5, v2, z117], axis=1)


def _mlp_tc(g, wcat, ba, wb, bb, n_edges, interpret=False):
  nblk = n_edges // _EDGE_BLK
  return pl.pallas_call(
      _mlp_body,
      grid=(nblk,),
      in_specs=[
          pl.BlockSpec((_EDGE_BLK, _D), lambda i: (i, 0)),
          pl.BlockSpec((_EDGE_BLK, _D), lambda i, _n=nblk: (i + _n, 0)),
          pl.BlockSpec((384, 256), lambda i: (0, 0)),
          pl.BlockSpec((1, 256), lambda i: (0, 0)),
          pl.BlockSpec((256, 2), lambda i: (0, 0)),
          pl.BlockSpec((1, 2), lambda i: (0, 0)),
      ],
      out_specs=pl.BlockSpec((_EDGE_BLK, 128), lambda i: (i, 0)),
      out_shape=jax.ShapeDtypeStruct((n_edges, 128), jnp.float32),
      interpret=interpret,
  )(g, g, wcat, ba, wb, bb)


def _gram_body(p0, p1, rnd, out):
  eps = 1e-6
  b = p0.shape[0]
  v = p0[...] + p1[...]
  v1 = v[:, 0:3]
  v2 = v[:, 8:11]

  def safe_norm(u):
    return jnp.sqrt(jnp.sum(u * u, axis=1, keepdims=True) + 1e-12)

  v1_norm = safe_norm(v1)
  mask1 = v1_norm > eps
  n1 = v1 / (v1_norm + eps)
  r = rnd[:, 0:3]
  r = r / safe_norm(r)
  n1 = jnp.where(mask1, n1, r)
  n2p = v2 - jnp.sum(n1 * v2, axis=1, keepdims=True) * n1
  n2_norm = safe_norm(n2p)
  mask2 = n2_norm > eps
  zc = jnp.zeros((b, 1), jnp.float32)
  oc = jnp.ones((b, 1), jnp.float32)
  default_orth = jnp.concatenate([-n1[:, 1:2], n1[:, 0:1], zc], axis=1)
  d_norm = safe_norm(default_orth)
  d_mask = d_norm > eps
  e3 = jnp.concatenate([zc, zc, oc], axis=1)
  default_orth = jnp.where(d_mask, default_orth / (d_norm + eps), e3)
  n2 = jnp.where(mask2, n2p / (n2_norm + eps), default_orth)
  n3 = jnp.concatenate([
      n1[:, 1:2] * n2[:, 2:3] - n1[:, 2:3] * n2[:, 1:2],
      n1[:, 2:3] * n2[:, 0:1] - n1[:, 0:1] * n2[:, 2:3],
      n1[:, 0:1] * n2[:, 1:2] - n1[:, 1:2] * n2[:, 0:1],
  ], axis=1)
  n3 = n3 / (safe_norm(n3) + eps)
  z7 = jnp.zeros((b, 7), jnp.float32)
  out[...] = jnp.concatenate([n1, n2, n3, z7], axis=1)


def _gram_tc(p0, p1, rnd16, interpret=False):
  n_pad = p0.shape[0]
  return pl.pallas_call(
      _gram_body,
      grid=(n_pad // _NODE_BLK,),
      in_specs=[
          pl.BlockSpec((_NODE_BLK, 128), lambda i: (i, 0)),
          pl.BlockSpec((_NODE_BLK, 128), lambda i: (i, 0)),
          pl.BlockSpec((_NODE_BLK, 16), lambda i: (i, 0)),
      ],
      out_specs=pl.BlockSpec((_NODE_BLK, 16), lambda i: (i, 0)),
      out_shape=jax.ShapeDtypeStruct((n_pad, 16), jnp.float32),
      interpret=interpret,
  )(p0, p1, rnd16)


def kernel(x, pos, edge_index, W1a, b1a, W1b, b1b, W2a, b2a, W2b, b2b):
  n, f = x.shape
  e = edge_index.shape[1]
  n_pad = ((n + 1023) // 1024) * 1024

  table = jnp.concatenate(
      [x, pos, jnp.zeros((n, _D - f - 3), jnp.float32)], axis=1)

  idx_flat = edge_index.reshape(2 * e)
  g = _gather_sc(table, idx_flat)

  wcat = jnp.concatenate([
      jnp.concatenate([W1a, W2a], axis=1),
      jnp.zeros((384 - (2 * f + 1), 256), jnp.float32),
  ], axis=0)
  ba = jnp.concatenate([b1a, b2a]).reshape(1, 256)
  wb = jnp.concatenate([
      jnp.concatenate([W1b, jnp.zeros_like(W1b)], axis=1),
      jnp.concatenate([jnp.zeros_like(W2b), W2b], axis=1),
  ], axis=0)
  bb = jnp.concatenate([b1b, b2b]).reshape(1, 2)

  vec = _mlp_tc(g, wcat, ba, wb, bb, e)

  c = _SCAT_C
  j_steps = e // (32 * c)
  col3 = edge_index[1].reshape(32, j_steps, c)
  zeros_acc = jnp.zeros((n_pad, 128), jnp.float32)
  partials = _scatter_add_sc(vec, col3, zeros_acc)

  rnd = jax.random.normal(jax.random.key(42), (n, 3), dtype=jnp.float32)
  rnd16 = jnp.concatenate(
      [rnd, jnp.ones((n, 13), jnp.float32)], axis=1)
  rnd16 = jnp.concatenate(
      [rnd16, jnp.ones((n_pad - n, 16), jnp.float32)], axis=0)
  res = _gram_tc(partials[0], partials[1], rnd16)
  frames = res[:n, :9].reshape(n, 3, 3).transpose(0, 2, 1)
  return frames


# speedup vs baseline: 4.3196x; 1.0592x over previous
import functools

import jax
import jax.numpy as jnp
from jax import lax
from jax.experimental import pallas as pl
from jax.experimental.pallas import tpu as pltpu
from jax.experimental.pallas import tpu_sc as plsc

_D = 256
_GATHER_W = 128
_EDGE_BLK = 1600
_SCAT_C = 80
_NODE_BLK = 1024


def _gather_sc(table, idx_flat):
  m = idx_flat.shape[0]
  d = table.shape[1]
  mesh = plsc.VectorSubcoreMesh(core_axis_name="core",
                                subcore_axis_name="subcore")
  idx2 = idx_flat.reshape(1, m)

  @functools.partial(
      pl.kernel,
      out_type=jax.ShapeDtypeStruct((m, d), table.dtype),
      mesh=mesh)
  def k(t_hbm, i_hbm, o_hbm):
    def body(i_vmem, o_vmem):
      pltpu.sync_copy(t_hbm.at[i_vmem.at[0]], o_vmem)

    pltpu.emit_pipeline(
        body,
        grid=(m // _GATHER_W,),
        in_specs=[pl.BlockSpec((1, _GATHER_W), lambda i: (0, i))],
        out_specs=[pl.BlockSpec((_GATHER_W, d), lambda i: (i, 0))],
        core_axis_name=("core", "subcore"),
        dimension_semantics=(pltpu.PARALLEL,),
    )(i_hbm, o_hbm)

  return k(table, idx2)


def _scatter_add_sc(vec, col3, zeros_acc):
  n_pad = zeros_acc.shape[0]
  j_steps = col3.shape[1]
  c = col3.shape[2]
  rows_per_tile = n_pad // 16
  mesh = plsc.VectorSubcoreMesh(core_axis_name="core",
                                subcore_axis_name="subcore")

  @functools.partial(
      pl.kernel,
      out_type=jax.ShapeDtypeStruct((2, n_pad, 128), jnp.float32),
      mesh=mesh,
      scratch_types=[
          pltpu.VMEM((c,), jnp.int32),
          pltpu.VMEM((c, 128), jnp.float32),
          pltpu.VMEM_SHARED((n_pad, 128), jnp.float32),
      ])
  def k(vec_hbm, col_hbm, z_hbm, out_hbm, idx_v, vec_v, acc_sh):
    cid = lax.axis_index("core")
    sid = lax.axis_index("subcore")
    wid = cid * 16 + sid

    @pl.when(sid == 0)
    def _():
      pltpu.sync_copy(z_hbm, acc_sh)

    plsc.subcore_barrier()

    @pl.loop(0, j_steps)
    def _(j):
      base = (wid * j_steps + j) * c
      pltpu.sync_copy(col_hbm.at[wid, j], idx_v)
      pltpu.sync_copy(vec_hbm.at[pl.ds(base, c)], vec_v)
      pltpu.sync_copy(vec_v, acc_sh.at[idx_v], add=True)

    plsc.subcore_barrier()
    pltpu.sync_copy(acc_sh.at[pl.ds(sid * rows_per_tile, rows_per_tile)],
                    out_hbm.at[cid, pl.ds(sid * rows_per_tile, rows_per_tile)])

  return k(vec, col3, zeros_acc)


def _omega(dist):
  p = 5.0
  dr = jnp.clip(dist / 4.5, 0.0, 1.0)
  dr2 = dr * dr
  dr5 = dr2 * dr2 * dr
  dr6 = dr5 * dr
  dr7 = dr6 * dr
  out = (1.0
         - (p + 1.0) * (p + 2.0) / 2.0 * dr5
         + p * (p + 2.0) * dr6
         - p * (p + 1.0) / 2.0 * dr7)
  return jnp.clip(out, 0.0, 1.0)


def _mlp_body(gr, gc, wcat, ba, wb, bb, out):
  xr = gr[:, :128]
  xc = gc[:, :128]
  pr = gr[:, 128:131]
  pc = gc[:, 128:131]
  d = pr - pc
  dist = jnp.sqrt(jnp.sum(d * d, axis=1, keepdims=True) + 1e-12)
  b = d.shape[0]
  xcat = jnp.concatenate([xr, xc, dist, jnp.zeros((b, 127), jnp.float32)],
                         axis=1)
  h = jnp.dot(xcat, wcat[...], preferred_element_type=jnp.float32) + ba[...]
  h = jax.nn.silu(h)
  mes = jnp.dot(h, wb[...], preferred_element_type=jnp.float32) + bb[...]
  coe = _omega(dist)
  nv = d / (dist + 1e-8)
  v1 = nv * coe * mes[:, 0:1]
  v2 = nv * coe * mes[:, 1:2]
  z5 = jnp.zeros((b, 5), jnp.float32)
  z117 = jnp.zeros((b, 117), jnp.float32)
  out[...] = jnp.concatenate([v1, z5, v2, z117], axis=1)


def _mlp_tc(g, wcat, ba, wb, bb, n_edges, interpret=False):
  nblk = n_edges // _EDGE_BLK
  return pl.pallas_call(
      _mlp_body,
      grid=(nblk,),
      in_specs=[
          pl.BlockSpec((_EDGE_BLK, _D), lambda i: (i, 0)),
          pl.BlockSpec((_EDGE_BLK, _D), lambda i, _n=nblk: (i + _n, 0)),
          pl.BlockSpec((384, 256), lambda i: (0, 0)),
          pl.BlockSpec((1, 256), lambda i: (0, 0)),
          pl.BlockSpec((256, 2), lambda i: (0, 0)),
          pl.BlockSpec((1, 2), lambda i: (0, 0)),
      ],
      out_specs=pl.BlockSpec((_EDGE_BLK, 128), lambda i: (i, 0)),
      out_shape=jax.ShapeDtypeStruct((n_edges, 128), jnp.float32),
      interpret=interpret,
  )(g, g, wcat, ba, wb, bb)


def _gram_body(*refs):
  eps = 1e-6
  parts = refs[:-2]
  rnd = refs[-2]
  out = refs[-1]
  b = parts[0].shape[0]
  v = parts[0][...]
  for pk in parts[1:]:
    v = v + pk[...]
  v1 = v[:, 0:3]
  v2 = v[:, 8:11]

  def safe_norm(u):
    return jnp.sqrt(jnp.sum(u * u, axis=1, keepdims=True) + 1e-12)

  v1_norm = safe_norm(v1)
  mask1 = v1_norm > eps
  n1 = v1 / (v1_norm + eps)
  r = rnd[:, 0:3]
  r = r / safe_norm(r)
  n1 = jnp.where(mask1, n1, r)
  n2p = v2 - jnp.sum(n1 * v2, axis=1, keepdims=True) * n1
  n2_norm = safe_norm(n2p)
  mask2 = n2_norm > eps
  zc = jnp.zeros((b, 1), jnp.float32)
  oc = jnp.ones((b, 1), jnp.float32)
  default_orth = jnp.concatenate([-n1[:, 1:2], n1[:, 0:1], zc], axis=1)
  d_norm = safe_norm(default_orth)
  d_mask = d_norm > eps
  e3 = jnp.concatenate([zc, zc, oc], axis=1)
  default_orth = jnp.where(d_mask, default_orth / (d_norm + eps), e3)
  n2 = jnp.where(mask2, n2p / (n2_norm + eps), default_orth)
  n3 = jnp.concatenate([
      n1[:, 1:2] * n2[:, 2:3] - n1[:, 2:3] * n2[:, 1:2],
      n1[:, 2:3] * n2[:, 0:1] - n1[:, 0:1] * n2[:, 2:3],
      n1[:, 0:1] * n2[:, 1:2] - n1[:, 1:2] * n2[:, 0:1],
  ], axis=1)
  n3 = n3 / (safe_norm(n3) + eps)
  z7 = jnp.zeros((b, 7), jnp.float32)
  out[...] = jnp.concatenate([n1, n2, n3, z7], axis=1)


def _gram_tc(parts, rnd16, interpret=False):
  n_pad = parts[0].shape[0]
  return pl.pallas_call(
      _gram_body,
      grid=(n_pad // _NODE_BLK,),
      in_specs=(
          [pl.BlockSpec((_NODE_BLK, 128), lambda i: (i, 0))] * len(parts)
          + [pl.BlockSpec((_NODE_BLK, 16), lambda i: (i, 0))]),
      out_specs=pl.BlockSpec((_NODE_BLK, 16), lambda i: (i, 0)),
      out_shape=jax.ShapeDtypeStruct((n_pad, 16), jnp.float32),
      interpret=interpret,
  )(*parts, rnd16)


def kernel(x, pos, edge_index, W1a, b1a, W1b, b1b, W2a, b2a, W2b, b2b):
  n, f = x.shape
  e = edge_index.shape[1]
  n_pad = ((n + 1023) // 1024) * 1024

  table = jnp.concatenate(
      [x, pos, jnp.zeros((n, _D - f - 3), jnp.float32)], axis=1)

  wcat = jnp.concatenate([
      jnp.concatenate([W1a, W2a], axis=1),
      jnp.zeros((384 - (2 * f + 1), 256), jnp.float32),
  ], axis=0)
  ba = jnp.concatenate([b1a, b2a]).reshape(1, 256)
  wb = jnp.concatenate([
      jnp.concatenate([W1b, jnp.zeros_like(W1b)], axis=1),
      jnp.concatenate([jnp.zeros_like(W2b), W2b], axis=1),
  ], axis=0)
  bb = jnp.concatenate([b1b, b2b]).reshape(1, 2)

  n_chunks = 2
  ec = e // n_chunks
  c = 40
  j_steps = ec // (32 * c)
  zeros_acc = jnp.zeros((n_pad, 128), jnp.float32)
  partial_list = []
  for k in range(n_chunks):
    rows_k = lax.dynamic_slice_in_dim(edge_index[0], k * ec, ec)
    cols_k = lax.dynamic_slice_in_dim(edge_index[1], k * ec, ec)
    idx_k = jnp.concatenate([rows_k, cols_k])
    g_k = _gather_sc(table, idx_k)
    vec_k = _mlp_tc(g_k, wcat, ba, wb, bb, ec)
    col3_k = cols_k.reshape(32, j_steps, c)
    partial_list.append(_scatter_add_sc(vec_k, col3_k, zeros_acc))

  rnd = jax.random.normal(jax.random.key(42), (n, 3), dtype=jnp.float32)
  rnd16 = jnp.concatenate(
      [rnd, jnp.ones((n, 13), jnp.float32)], axis=1)
  rnd16 = jnp.concatenate(
      [rnd16, jnp.ones((n_pad - n, 16), jnp.float32)], axis=0)
  p_all = [p[i] for p in partial_list for i in range(2)]
  res = _gram_tc(p_all, rnd16)
  frames = res[:n, :9].reshape(n, 3, 3).transpose(0, 2, 1)
  return frames


# speedup vs baseline: 4.3374x; 1.0041x over previous
import functools

import jax
import jax.numpy as jnp
from jax import lax
from jax.experimental import pallas as pl
from jax.experimental.pallas import tpu as pltpu
from jax.experimental.pallas import tpu_sc as plsc

_D = 256
_GATHER_W = 128
_EDGE_BLK = 3200
_SCAT_C = 80
_NODE_BLK = 1024


def _gather_sc(table, idx_flat):
  m = idx_flat.shape[0]
  d = table.shape[1]
  mesh = plsc.VectorSubcoreMesh(core_axis_name="core",
                                subcore_axis_name="subcore")
  idx2 = idx_flat.reshape(1, m)

  @functools.partial(
      pl.kernel,
      out_type=jax.ShapeDtypeStruct((m, d), table.dtype),
      mesh=mesh)
  def k(t_hbm, i_hbm, o_hbm):
    def body(i_vmem, o_vmem):
      pltpu.sync_copy(t_hbm.at[i_vmem.at[0]], o_vmem)

    pltpu.emit_pipeline(
        body,
        grid=(m // _GATHER_W,),
        in_specs=[pl.BlockSpec((1, _GATHER_W), lambda i: (0, i))],
        out_specs=[pl.BlockSpec((_GATHER_W, d), lambda i: (i, 0))],
        core_axis_name=("core", "subcore"),
        dimension_semantics=(pltpu.PARALLEL,),
    )(i_hbm, o_hbm)

  return k(table, idx2)


def _scatter_add_sc(vec, col3, zeros_acc):
  n_pad = zeros_acc.shape[0]
  j_steps = col3.shape[1]
  c = col3.shape[2]
  rows_per_tile = n_pad // 16
  mesh = plsc.VectorSubcoreMesh(core_axis_name="core",
                                subcore_axis_name="subcore")

  @functools.partial(
      pl.kernel,
      out_type=jax.ShapeDtypeStruct((2, n_pad, 128), jnp.float32),
      mesh=mesh,
      scratch_types=[
          pltpu.VMEM((c,), jnp.int32),
          pltpu.VMEM((c, 128), jnp.float32),
          pltpu.VMEM_SHARED((n_pad, 128), jnp.float32),
      ])
  def k(vec_hbm, col_hbm, z_hbm, out_hbm, idx_v, vec_v, acc_sh):
    cid = lax.axis_index("core")
    sid = lax.axis_index("subcore")
    wid = cid * 16 + sid

    @pl.when(sid == 0)
    def _():
      pltpu.sync_copy(z_hbm, acc_sh)

    plsc.subcore_barrier()

    @pl.loop(0, j_steps)
    def _(j):
      base = (wid * j_steps + j) * c
      pltpu.sync_copy(col_hbm.at[wid, j], idx_v)
      pltpu.sync_copy(vec_hbm.at[pl.ds(base, c)], vec_v)
      pltpu.sync_copy(vec_v, acc_sh.at[idx_v], add=True)

    plsc.subcore_barrier()
    pltpu.sync_copy(acc_sh.at[pl.ds(sid * rows_per_tile, rows_per_tile)],
                    out_hbm.at[cid, pl.ds(sid * rows_per_tile, rows_per_tile)])

  return k(vec, col3, zeros_acc)


def _omega(dist):
  p = 5.0
  dr = jnp.clip(dist / 4.5, 0.0, 1.0)
  dr2 = dr * dr
  dr5 = dr2 * dr2 * dr
  dr6 = dr5 * dr
  dr7 = dr6 * dr
  out = (1.0
         - (p + 1.0) * (p + 2.0) / 2.0 * dr5
         + p * (p + 2.0) * dr6
         - p * (p + 1.0) / 2.0 * dr7)
  return jnp.clip(out, 0.0, 1.0)


def _mlp_body(gr, gc, wcat, ba, wb, bb, out):
  xr = gr[:, :128]
  xc = gc[:, :128]
  pr = gr[:, 128:131]
  pc = gc[:, 128:131]
  d = pr - pc
  dist = jnp.sqrt(jnp.sum(d * d, axis=1, keepdims=True) + 1e-12)
  b = d.shape[0]
  xcat = jnp.concatenate([xr, xc, dist, jnp.zeros((b, 127), jnp.float32)],
                         axis=1)
  h = jnp.dot(xcat, wcat[...], preferred_element_type=jnp.float32) + ba[...]
  h = jax.nn.silu(h)
  mes = jnp.dot(h, wb[...], preferred_element_type=jnp.float32) + bb[...]
  coe = _omega(dist)
  nv = d / (dist + 1e-8)
  v1 = nv * coe * mes[:, 0:1]
  v2 = nv * coe * mes[:, 1:2]
  z5 = jnp.zeros((b, 5), jnp.float32)
  z117 = jnp.zeros((b, 117), jnp.float32)
  out[...] = jnp.concatenate([v1, z5, v2, z117], axis=1)


def _mlp_tc(g, wcat, ba, wb, bb, n_edges, interpret=False):
  nblk = n_edges // _EDGE_BLK
  return pl.pallas_call(
      _mlp_body,
      grid=(nblk,),
      in_specs=[
          pl.BlockSpec((_EDGE_BLK, _D), lambda i: (i, 0)),
          pl.BlockSpec((_EDGE_BLK, _D), lambda i, _n=nblk: (i + _n, 0)),
          pl.BlockSpec((384, 256), lambda i: (0, 0)),
          pl.BlockSpec((1, 256), lambda i: (0, 0)),
          pl.BlockSpec((256, 2), lambda i: (0, 0)),
          pl.BlockSpec((1, 2), lambda i: (0, 0)),
      ],
      out_specs=pl.BlockSpec((_EDGE_BLK, 128), lambda i: (i, 0)),
      out_shape=jax.ShapeDtypeStruct((n_edges, 128), jnp.float32),
      interpret=interpret,
  )(g, g, wcat, ba, wb, bb)


def _gram_body(*refs):
  eps = 1e-6
  parts = refs[:-2]
  rnd = refs[-2]
  out = refs[-1]
  b = parts[0].shape[0]
  v = parts[0][...]
  for pk in parts[1:]:
    v = v + pk[...]
  v1 = v[:, 0:3]
  v2 = v[:, 8:11]

  def safe_norm(u):
    return jnp.sqrt(jnp.sum(u * u, axis=1, keepdims=True) + 1e-12)

  v1_norm = safe_norm(v1)
  mask1 = v1_norm > eps
  n1 = v1 / (v1_norm + eps)
  r = rnd[:, 0:3]
  r = r / safe_norm(r)
  n1 = jnp.where(mask1, n1, r)
  n2p = v2 - jnp.sum(n1 * v2, axis=1, keepdims=True) * n1
  n2_norm = safe_norm(n2p)
  mask2 = n2_norm > eps
  zc = jnp.zeros((b, 1), jnp.float32)
  oc = jnp.ones((b, 1), jnp.float32)
  default_orth = jnp.concatenate([-n1[:, 1:2], n1[:, 0:1], zc], axis=1)
  d_norm = safe_norm(default_orth)
  d_mask = d_norm > eps
  e3 = jnp.concatenate([zc, zc, oc], axis=1)
  default_orth = jnp.where(d_mask, default_orth / (d_norm + eps), e3)
  n2 = jnp.where(mask2, n2p / (n2_norm + eps), default_orth)
  n3 = jnp.concatenate([
      n1[:, 1:2] * n2[:, 2:3] - n1[:, 2:3] * n2[:, 1:2],
      n1[:, 2:3] * n2[:, 0:1] - n1[:, 0:1] * n2[:, 2:3],
      n1[:, 0:1] * n2[:, 1:2] - n1[:, 1:2] * n2[:, 0:1],
  ], axis=1)
  n3 = n3 / (safe_norm(n3) + eps)
  z7 = jnp.zeros((b, 7), jnp.float32)
  out[...] = jnp.concatenate([n1, n2, n3, z7], axis=1)


def _gram_tc(parts, rnd16, interpret=False):
  n_pad = parts[0].shape[0]
  return pl.pallas_call(
      _gram_body,
      grid=(n_pad // _NODE_BLK,),
      in_specs=(
          [pl.BlockSpec((_NODE_BLK, 128), lambda i: (i, 0))] * len(parts)
          + [pl.BlockSpec((_NODE_BLK, 16), lambda i: (i, 0))]),
      out_specs=pl.BlockSpec((_NODE_BLK, 16), lambda i: (i, 0)),
      out_shape=jax.ShapeDtypeStruct((n_pad, 16), jnp.float32),
      interpret=interpret,
  )(*parts, rnd16)


def kernel(x, pos, edge_index, W1a, b1a, W1b, b1b, W2a, b2a, W2b, b2b):
  n, f = x.shape
  e = edge_index.shape[1]
  n_pad = ((n + 1023) // 1024) * 1024

  table = jnp.concatenate(
      [x, pos, jnp.zeros((n, _D - f - 3), jnp.float32)], axis=1)

  wcat = jnp.concatenate([
      jnp.concatenate([W1a, W2a], axis=1),
      jnp.zeros((384 - (2 * f + 1), 256), jnp.float32),
  ], axis=0)
  ba = jnp.concatenate([b1a, b2a]).reshape(1, 256)
  wb = jnp.concatenate([
      jnp.concatenate([W1b, jnp.zeros_like(W1b)], axis=1),
      jnp.concatenate([jnp.zeros_like(W2b), W2b], axis=1),
  ], axis=0)
  bb = jnp.concatenate([b1b, b2b]).reshape(1, 2)

  n_chunks = 2
  ec = e // n_chunks
  c = 40
  j_steps = ec // (32 * c)
  zeros_acc = jnp.zeros((n_pad, 128), jnp.float32)
  partial_list = []
  for k in range(n_chunks):
    rows_k = lax.dynamic_slice_in_dim(edge_index[0], k * ec, ec)
    cols_k = lax.dynamic_slice_in_dim(edge_index[1], k * ec, ec)
    idx_k = jnp.concatenate([rows_k, cols_k])
    g_k = _gather_sc(table, idx_k)
    vec_k = _mlp_tc(g_k, wcat, ba, wb, bb, ec)
    col3_k = cols_k.reshape(32, j_steps, c)
    partial_list.append(_scatter_add_sc(vec_k, col3_k, zeros_acc))

  rnd = jax.random.normal(jax.random.key(42), (n, 3), dtype=jnp.float32)
  rnd16 = jnp.concatenate(
      [rnd, jnp.ones((n, 13), jnp.float32)], axis=1)
  rnd16 = jnp.concatenate(
      [rnd16, jnp.ones((n_pad - n, 16), jnp.float32)], axis=0)
  p_all = [p[i] for p in partial_list for i in range(2)]
  res = _gram_tc(p_all, rnd16)
  frames = res[:n, :9].reshape(n, 3, 3).transpose(0, 2, 1)
  return frames
